# baseline (speedup 1.0000x reference)
import functools

import jax
import jax.numpy as jnp
from jax import lax
from jax.experimental import pallas as pl
from jax.experimental.pallas import tpu as pltpu
from jax.experimental.pallas import tpu_sc as plsc

D = 32
L = 16
NW = 32
CHUNK = 128


@functools.lru_cache(maxsize=None)
def _build(B):
    BPW = B // NW
    NCH = BPW // CHUNK
    mesh = plsc.VectorSubcoreMesh(core_axis_name="c", subcore_axis_name="s")

    @functools.partial(
        pl.kernel,
        mesh=mesh,
        compiler_params=pltpu.CompilerParams(
            use_tc_tiling_on_sc=False, needs_layout_passes=False),
        out_type=jax.ShapeDtypeStruct((B,), jnp.float32),
        scratch_types=[
            pltpu.VMEM((NCH, CHUNK), jnp.int32),
            pltpu.VMEM((NCH, CHUNK), jnp.int32),
            pltpu.VMEM((BPW, D), jnp.float32),
            pltpu.VMEM((BPW, D), jnp.float32),
            pltpu.VMEM((BPW,), jnp.float32),
            pltpu.VMEM((BPW,), jnp.float32),
            pltpu.VMEM((L,), jnp.float32),
            pltpu.VMEM((BPW,), jnp.float32),
            pltpu.SemaphoreType.DMA,
        ],
    )
    def sc_kernel(ui_hbm, ii_hbm, ue_hbm, ie_hbm, ub_hbm, ib_hbm, gb_hbm,
                  out_hbm, ui_v, ii_v, ue_v, ie_v, ub_v, ib_v, gb_v, pred_v,
                  sem):
        wid = lax.axis_index("s") * 2 + lax.axis_index("c")
        base = wid * BPW

        pltpu.sync_copy(ui_hbm.at[pl.ds(wid * NCH, NCH)], ui_v)
        pltpu.sync_copy(ii_hbm.at[pl.ds(wid * NCH, NCH)], ii_v)
        pltpu.sync_copy(gb_hbm, gb_v)

        ub_flat = ub_hbm.at[0]
        ib_flat = ib_hbm.at[0]

        copies = []
        for j in range(NCH):
            sl = pl.ds(j * CHUNK, CHUNK)
            copies.append(pltpu.async_copy(ue_hbm.at[ui_v.at[j]], ue_v.at[sl], sem))
            copies.append(pltpu.async_copy(ie_hbm.at[ii_v.at[j]], ie_v.at[sl], sem))
            copies.append(pltpu.async_copy(ub_flat.at[ui_v.at[j]], ub_v.at[sl], sem))
            copies.append(pltpu.async_copy(ib_flat.at[ii_v.at[j]], ib_v.at[sl], sem))
        for c in copies:
            c.wait()

        gbs = gb_v[...]

        def body(g, carry):
            row0 = g * L
            sl = pl.ds(row0, L)
            riota = lax.iota(jnp.int32, L) + row0
            acc = jnp.zeros((L,), jnp.float32)
            for j in range(D):
                cj = jnp.full((L,), j, jnp.int32)
                u = plsc.load_gather(ue_v, [riota, cj])
                t = plsc.load_gather(ie_v, [riota, cj])
                acc = acc + jnp.maximum(u, 0.0) * jnp.maximum(t, 0.0)
            pred_v[sl] = acc + ub_v[sl] + ib_v[sl] + gbs
            return carry

        lax.fori_loop(0, BPW // L, body, 0)
        pltpu.sync_copy(pred_v, out_hbm.at[pl.ds(base, BPW)])

    return sc_kernel


def kernel(user_indices, item_indices, user_emb, item_emb, user_bias,
           item_bias, global_bias):
    B = user_indices.shape[0]
    ui = user_indices.reshape(B // CHUNK, CHUNK)
    ii = item_indices.reshape(B // CHUNK, CHUNK)
    gb = jnp.broadcast_to(global_bias, (L,))
    return _build(B)(ui, ii, user_emb, item_emb, user_bias.T, item_bias.T, gb)


# speedup vs baseline: 1.0005x; 1.0005x over previous
import functools

import jax
import jax.numpy as jnp
from jax import lax
from jax.experimental import pallas as pl
from jax.experimental.pallas import tpu as pltpu
from jax.experimental.pallas import tpu_sc as plsc

D = 32
L = 16
NW = 32
CHUNK = 128
W = 128
HALF = 256


@functools.lru_cache(maxsize=None)
def _build_bias(B):
    BPW = B // NW
    NCH = BPW // CHUNK
    mesh = plsc.VectorSubcoreMesh(core_axis_name="c", subcore_axis_name="s")

    @functools.partial(
        pl.kernel,
        mesh=mesh,
        compiler_params=pltpu.CompilerParams(needs_layout_passes=False),
        out_type=jax.ShapeDtypeStruct((B,), jnp.float32),
        scratch_types=[
            pltpu.VMEM((NCH, CHUNK), jnp.int32),
            pltpu.VMEM((NCH, CHUNK), jnp.int32),
            pltpu.VMEM((HALF, W), jnp.float32),
            pltpu.VMEM((HALF, W), jnp.float32),
            pltpu.VMEM((L,), jnp.float32),
            pltpu.VMEM((BPW,), jnp.float32),
            pltpu.SemaphoreType.DMA,
        ],
    )
    def bias_kernel(ui_hbm, ii_hbm, ubt_hbm, ibt_hbm, gb_hbm, out_hbm,
                    ui_v, ii_v, uw_v, iw_v, gb_v, part_v, sem):
        wid = lax.axis_index("s") * 2 + lax.axis_index("c")
        base = wid * BPW
        pltpu.sync_copy(ui_hbm.at[pl.ds(wid * NCH, NCH)], ui_v)
        pltpu.sync_copy(ii_hbm.at[pl.ds(wid * NCH, NCH)], ii_v)
        pltpu.sync_copy(gb_hbm, gb_v)
        gbs = gb_v[...]

        for h in range(BPW // HALF):
            k0 = h * HALF

            def fire(g, carry):
                kk = k0 + g * L
                jr = kk // CHUNK
                jo = kk % CHUNK
                uvec = ui_v[jr, pl.ds(jo, L)]
                ivec = ii_v[jr, pl.ds(jo, L)]
                for l in range(L):
                    uo = pl.multiple_of(
                        lax.shift_left(lax.shift_right_logical(uvec[l], 7), 7), W)
                    io = pl.multiple_of(
                        lax.shift_left(lax.shift_right_logical(ivec[l], 7), 7), W)
                    kl = (g * L + l)
                    pltpu.async_copy(ubt_hbm.at[:, pl.ds(uo, W)],
                                     uw_v.at[pl.ds(kl, 1)], sem)
                    pltpu.async_copy(ibt_hbm.at[:, pl.ds(io, W)],
                                     iw_v.at[pl.ds(kl, 1)], sem)
                return carry

            lax.fori_loop(0, HALF // L, fire, 0)

            def drain(g, carry):
                pltpu.make_async_copy(ubt_hbm.at[:, pl.ds(0, W)],
                                      uw_v.at[pl.ds(0, 1)], sem).wait()
                pltpu.make_async_copy(ubt_hbm.at[:, pl.ds(0, W)],
                                      iw_v.at[pl.ds(0, 1)], sem).wait()
                return carry

            lax.fori_loop(0, HALF, drain, 0)

            def sel(g, carry):
                kk = k0 + g * L
                jr = kk // CHUNK
                jo = kk % CHUNK
                uvec = ui_v[jr, pl.ds(jo, L)]
                ivec = ii_v[jr, pl.ds(jo, L)]
                riota = lax.iota(jnp.int32, L) + g * L
                ub = plsc.load_gather(uw_v, [riota, jnp.bitwise_and(uvec, W - 1)])
                ib = plsc.load_gather(iw_v, [riota, jnp.bitwise_and(ivec, W - 1)])
                part_v[pl.ds(kk, L)] = ub + ib + gbs
                return carry

            lax.fori_loop(0, HALF // L, sel, 0)

        pltpu.sync_copy(part_v, out_hbm.at[pl.ds(base, BPW)])

    return bias_kernel


@functools.lru_cache(maxsize=None)
def _build_main(B):
    BPW = B // NW
    NCH = BPW // CHUNK
    mesh = plsc.VectorSubcoreMesh(core_axis_name="c", subcore_axis_name="s")

    @functools.partial(
        pl.kernel,
        mesh=mesh,
        compiler_params=pltpu.CompilerParams(
            use_tc_tiling_on_sc=False, needs_layout_passes=False),
        out_type=jax.ShapeDtypeStruct((B,), jnp.float32),
        scratch_types=[
            pltpu.VMEM((NCH, CHUNK), jnp.int32),
            pltpu.VMEM((NCH, CHUNK), jnp.int32),
            pltpu.VMEM((BPW, D), jnp.float32),
            pltpu.VMEM((BPW, D), jnp.float32),
            pltpu.VMEM((BPW,), jnp.float32),
            pltpu.VMEM((BPW,), jnp.float32),
            pltpu.SemaphoreType.DMA,
        ],
    )
    def main_kernel(ui_hbm, ii_hbm, ue_hbm, ie_hbm, bias_hbm, out_hbm,
                    ui_v, ii_v, ue_v, ie_v, bias_v, pred_v, sem):
        wid = lax.axis_index("s") * 2 + lax.axis_index("c")
        base = wid * BPW
        pltpu.sync_copy(ui_hbm.at[pl.ds(wid * NCH, NCH)], ui_v)
        pltpu.sync_copy(ii_hbm.at[pl.ds(wid * NCH, NCH)], ii_v)
        pltpu.sync_copy(bias_hbm.at[pl.ds(base, BPW)], bias_v)

        copies = []
        for j in range(NCH):
            sl = pl.ds(j * CHUNK, CHUNK)
            copies.append(pltpu.async_copy(ue_hbm.at[ui_v.at[j]], ue_v.at[sl], sem))
            copies.append(pltpu.async_copy(ie_hbm.at[ii_v.at[j]], ie_v.at[sl], sem))
        for c in copies:
            c.wait()

        def body(g, carry):
            row0 = g * L
            sl = pl.ds(row0, L)
            riota = lax.iota(jnp.int32, L) + row0
            acc = jnp.zeros((L,), jnp.float32)
            for j in range(D):
                cj = jnp.full((L,), j, jnp.int32)
                u = plsc.load_gather(ue_v, [riota, cj])
                t = plsc.load_gather(ie_v, [riota, cj])
                acc = acc + jnp.maximum(u, 0.0) * jnp.maximum(t, 0.0)
            pred_v[sl] = acc + bias_v[sl]
            return carry

        lax.fori_loop(0, BPW // L, body, 0)
        pltpu.sync_copy(pred_v, out_hbm.at[pl.ds(base, BPW)])

    return main_kernel


def kernel(user_indices, item_indices, user_emb, item_emb, user_bias,
           item_bias, global_bias):
    B = user_indices.shape[0]
    ui = user_indices.reshape(B // CHUNK, CHUNK)
    ii = item_indices.reshape(B // CHUNK, CHUNK)
    gb = jnp.broadcast_to(global_bias, (L,))
    bias_part = _build_bias(B)(ui, ii, user_bias.T, item_bias.T, gb)
    return _build_main(B)(ui, ii, user_emb, item_emb, bias_part)
